# baseline (speedup 1.0000x reference)
import jax
import jax.numpy as jnp
from jax import lax
from jax.experimental import pallas as pl
from jax.experimental.pallas import tpu as pltpu
from jax.experimental.pallas import tpu_sc as plsc

N_EMBD = 2048
E = 16
TOPK = 2
NT = 4 * 4096
NC, NS = 2, 16
NW = NC * NS
TPW = NT // NW
LANES = 16
CHUNKS = TPW // LANES


def _gate_matmul_kernel(x_ref, w_ref, ns_ref, nw_ref, out_ref):
    logits = lax.dot_general(
        x_ref[...], w_ref[...],
        dimension_numbers=(((1,), (1,)), ((), ())),
        preferred_element_type=jnp.float32)
    out_ref[...] = logits + nw_ref[...] * ns_ref[...]


def _routing_kernel(lg_hbm, probs_hbm, idx_hbm, val_hbm,
                    lt_v, probs_v, idx_v, val_v):
    wid = lax.axis_index("s") * NC + lax.axis_index("c")
    base = wid * TPW
    pltpu.sync_copy(lg_hbm.at[pl.ds(base * E, TPW * E)], lt_v)
    lanes = lax.iota(jnp.int32, LANES)

    def chunk(c, carry):
        off = c * LANES
        tok = off + lanes
        prow = tok * E
        rows = [plsc.load_gather(lt_v, [prow + e]) for e in range(E)]
        best = rows[0]
        bidx = jnp.zeros((LANES,), jnp.int32)
        best2 = jnp.full((LANES,), -jnp.inf, jnp.float32)
        b2idx = jnp.zeros((LANES,), jnp.int32)
        for e in range(1, E):
            v = rows[e]
            gt1 = v > best
            gt2 = v > best2
            e_i = jnp.full((LANES,), e, jnp.int32)
            b2idx = jnp.where(gt1, bidx, jnp.where(gt2, e_i, b2idx))
            best2 = jnp.where(gt1, best, jnp.where(gt2, v, best2))
            bidx = jnp.where(gt1, e_i, bidx)
            best = jnp.where(gt1, v, best)
        ed = jnp.exp(best2 - best)
        denom = 1.0 + ed
        p1 = 1.0 / denom
        p2 = ed / denom
        for e in range(E):
            row = (jnp.where(bidx == e, p1, 0.0)
                   + jnp.where(b2idx == e, p2, 0.0))
            plsc.store_scatter(probs_v, [prow + e], row)
        krow = tok * TOPK
        plsc.store_scatter(idx_v, [krow], bidx)
        plsc.store_scatter(idx_v, [krow + 1], b2idx)
        plsc.store_scatter(val_v, [krow], best)
        plsc.store_scatter(val_v, [krow + 1], best2)
        return carry

    lax.fori_loop(0, CHUNKS, chunk, 0)
    pltpu.sync_copy(probs_v, probs_hbm.at[pl.ds(base * E, TPW * E)])
    pltpu.sync_copy(idx_v, idx_hbm.at[pl.ds(base * TOPK, TPW * TOPK)])
    pltpu.sync_copy(val_v, val_hbm.at[pl.ds(base * TOPK, TPW * TOPK)])


def _make_routing_call():
    mesh = plsc.VectorSubcoreMesh(
        core_axis_name="c", subcore_axis_name="s",
        num_cores=NC, num_subcores=NS)
    return pl.kernel(
        _routing_kernel,
        out_type=[
            jax.ShapeDtypeStruct((NT * E,), jnp.float32),
            jax.ShapeDtypeStruct((NT * TOPK,), jnp.int32),
            jax.ShapeDtypeStruct((NT * TOPK,), jnp.float32),
        ],
        mesh=mesh,
        scratch_types=[
            pltpu.VMEM((TPW * E,), jnp.float32),
            pltpu.VMEM((TPW * E,), jnp.float32),
            pltpu.VMEM((TPW * TOPK,), jnp.int32),
            pltpu.VMEM((TPW * TOPK,), jnp.float32),
        ],
        compiler_params=pltpu.CompilerParams(needs_layout_passes=False),
    )


def kernel(x, W_gate, noise_weight, noise):
    x2 = x.reshape(NT, N_EMBD)
    noise2 = noise.reshape(NT, E)
    nw2 = noise_weight.reshape(1, E)

    logits = pl.pallas_call(
        _gate_matmul_kernel,
        grid=(NW,),
        in_specs=[
            pl.BlockSpec((TPW, N_EMBD), lambda w: (w, 0)),
            pl.BlockSpec((E, N_EMBD), lambda w: (0, 0)),
            pl.BlockSpec((TPW, E), lambda w: (w, 0)),
            pl.BlockSpec((1, E), lambda w: (0, 0)),
        ],
        out_specs=pl.BlockSpec((TPW, E), lambda w: (w, 0)),
        out_shape=jax.ShapeDtypeStruct((NT, E), jnp.float32),
        compiler_params=pltpu.CompilerParams(
            dimension_semantics=("arbitrary",)),
    )(x2, W_gate, noise2, nw2)

    probs, idx, vals = _make_routing_call()(logits.reshape(NT * E))
    B, S = x.shape[0], x.shape[1]
    return (probs.reshape(B, S, E),
            idx.reshape(B, S, TOPK),
            vals.reshape(B, S, TOPK))


# speedup vs baseline: 1.7428x; 1.7428x over previous
import jax
import jax.numpy as jnp
from jax import lax
from jax.experimental import pallas as pl
from jax.experimental.pallas import tpu as pltpu
from jax.experimental.pallas import tpu_sc as plsc

N_EMBD = 2048
E = 16
TOPK = 2
B, S = 4, 4096
NT = B * S
NC, NS = 2, 16
NW = NC * NS
TPW = NT // NW
TPB = S // TPW
LANES = 16
CHUNKS = TPW // LANES


def _gate_matmul_kernel(x_ref, w_ref, nt_ref, nw_ref, out_ref):
    logits = lax.dot_general(
        w_ref[...], x_ref[...],
        dimension_numbers=(((1,), (1,)), ((), ())),
        preferred_element_type=jnp.float32)
    out_ref[...] = (logits + nw_ref[...] * nt_ref[...])[None]


def _routing_kernel(lg_hbm, probs_hbm, idx_hbm, val_hbm,
                    lt_v, probs_v, idx_v, val_v):
    wid = lax.axis_index("s") * NC + lax.axis_index("c")
    b = wid // TPB
    s0 = (wid % TPB) * TPW
    pltpu.sync_copy(lg_hbm.at[wid], lt_v)

    def chunk(c, carry):
        off = c * LANES
        rows = [lt_v[e, pl.ds(off, LANES)] for e in range(E)]
        best = rows[0]
        bidx = jnp.zeros((LANES,), jnp.int32)
        best2 = jnp.full((LANES,), -jnp.inf, jnp.float32)
        b2idx = jnp.zeros((LANES,), jnp.int32)
        for e in range(1, E):
            v = rows[e]
            gt1 = v > best
            gt2 = v > best2
            e_i = jnp.full((LANES,), e, jnp.int32)
            b2idx = jnp.where(gt1, bidx, jnp.where(gt2, e_i, b2idx))
            best2 = jnp.where(gt1, best, jnp.where(gt2, v, best2))
            bidx = jnp.where(gt1, e_i, bidx)
            best = jnp.where(gt1, v, best)
        ed = jnp.exp(best2 - best)
        denom = 1.0 + ed
        p1 = 1.0 / denom
        p2 = ed / denom
        for e in range(E):
            probs_v[e, pl.ds(off, LANES)] = (
                jnp.where(bidx == e, p1, 0.0)
                + jnp.where(b2idx == e, p2, 0.0))
        idx_v[0, pl.ds(off, LANES)] = bidx
        idx_v[1, pl.ds(off, LANES)] = b2idx
        val_v[0, pl.ds(off, LANES)] = best
        val_v[1, pl.ds(off, LANES)] = best2
        return carry

    lax.fori_loop(0, CHUNKS, chunk, 0)
    for e in range(E):
        pltpu.sync_copy(probs_v.at[e], probs_hbm.at[b, e, pl.ds(s0, TPW)])
    for k in range(TOPK):
        pltpu.sync_copy(idx_v.at[k], idx_hbm.at[b, k, pl.ds(s0, TPW)])
        pltpu.sync_copy(val_v.at[k], val_hbm.at[b, k, pl.ds(s0, TPW)])


def _make_routing_call():
    mesh = plsc.VectorSubcoreMesh(
        core_axis_name="c", subcore_axis_name="s",
        num_cores=NC, num_subcores=NS)
    return pl.kernel(
        _routing_kernel,
        out_type=[
            jax.ShapeDtypeStruct((B, E, S), jnp.float32),
            jax.ShapeDtypeStruct((B, TOPK, S), jnp.int32),
            jax.ShapeDtypeStruct((B, TOPK, S), jnp.float32),
        ],
        mesh=mesh,
        scratch_types=[
            pltpu.VMEM((E, TPW), jnp.float32),
            pltpu.VMEM((E, TPW), jnp.float32),
            pltpu.VMEM((TOPK, TPW), jnp.int32),
            pltpu.VMEM((TOPK, TPW), jnp.float32),
        ],
        compiler_params=pltpu.CompilerParams(needs_layout_passes=False),
    )


def kernel(x, W_gate, noise_weight, noise):
    x2 = x.reshape(NT, N_EMBD)
    noise_t = noise.reshape(NT, E).T
    nw2 = noise_weight.reshape(E, 1)

    logits3 = pl.pallas_call(
        _gate_matmul_kernel,
        grid=(NW,),
        in_specs=[
            pl.BlockSpec((TPW, N_EMBD), lambda w: (w, 0)),
            pl.BlockSpec((E, N_EMBD), lambda w: (0, 0)),
            pl.BlockSpec((E, TPW), lambda w: (0, w)),
            pl.BlockSpec((E, 1), lambda w: (0, 0)),
        ],
        out_specs=pl.BlockSpec((1, E, TPW), lambda w: (w, 0, 0)),
        out_shape=jax.ShapeDtypeStruct((NW, E, TPW), jnp.float32),
        compiler_params=pltpu.CompilerParams(
            dimension_semantics=("arbitrary",)),
    )(x2, W_gate, noise_t, nw2)

    probs, idx, vals = _make_routing_call()(logits3)
    return (probs.transpose(0, 2, 1),
            idx.transpose(0, 2, 1),
            vals.transpose(0, 2, 1))
